# Initial kernel scaffold; baseline (speedup 1.0000x reference)
import jax
import jax.numpy as jnp
from jax.experimental import pallas as pl


def kernel(x, codebook):
    raise NotImplementedError("write your pallas kernel here")


# speedup vs baseline: 1.1636x; 1.1636x over previous
import functools

import jax
import jax.numpy as jnp
from jax import lax
from jax.experimental import pallas as pl
from jax.experimental.pallas import tpu as pltpu
from jax.experimental.pallas import tpu_sc as plsc

_K = 8192
_D = 32
_N = 8192
_NB = 256
_NSTEPS = _N // _NB

_SC_CORES = 2
_SC_SUBCORES = 16
_SC_WORKERS = _SC_CORES * _SC_SUBCORES
_ROWS_PER_W = _N // _SC_WORKERS
_IDX_CHUNK = 128


def _argmin_body(z_ref, cbt2_ref, idx_ref):
    z = z_ref[...]
    cbt2 = cbt2_ref[...]
    mm = lax.dot_general(z, cbt2, (((1,), (0,)), ((), ())),
                         preferred_element_type=jnp.float32)
    zsq = jnp.sum(z * z, axis=1, keepdims=True)
    cb = cbt2 * (-0.5)
    cbsq = jnp.sum(cb * cb, axis=0, keepdims=True)
    dist = (zsq + mm) + cbsq
    m = jnp.min(dist, axis=1, keepdims=True)
    iota = lax.broadcasted_iota(jnp.int32, dist.shape, 1)
    idx = jnp.min(jnp.where(dist == m, iota, jnp.int32(_K)), axis=1)
    idx_ref[0, 0, :] = idx


def _tc_argmin(z2d, cbt2):
    out = pl.pallas_call(
        _argmin_body,
        grid=(_NSTEPS,),
        in_specs=[
            pl.BlockSpec((_NB, _D), lambda i: (i, 0)),
            pl.BlockSpec((_D, _K), lambda i: (0, 0)),
        ],
        out_specs=pl.BlockSpec((1, 1, _NB), lambda i: (i, 0, 0)),
        out_shape=jax.ShapeDtypeStruct((_NSTEPS, 1, _NB), jnp.int32),
    )(z2d, cbt2)
    return out.reshape(_N)


_GATHER_W = 128


@functools.partial(
    pl.kernel,
    out_type=jax.ShapeDtypeStruct((_N, _GATHER_W), jnp.float32),
    mesh=plsc.VectorSubcoreMesh(core_axis_name="c", subcore_axis_name="s"),
    scratch_types=[
        pltpu.VMEM((_ROWS_PER_W // _IDX_CHUNK, _IDX_CHUNK), jnp.int32),
        pltpu.VMEM((_ROWS_PER_W, _GATHER_W), jnp.float32),
        pltpu.SemaphoreType.DMA,
    ],
)
def _sc_gather(table_hbm, idx_hbm, out_hbm, idx_v, rows_v, sem):
    wid = lax.axis_index("s") * _SC_CORES + lax.axis_index("c")
    base = wid * _ROWS_PER_W
    nchunks = _ROWS_PER_W // _IDX_CHUNK
    for c in range(nchunks):
        pltpu.sync_copy(idx_hbm.at[pl.ds(base + c * _IDX_CHUNK, _IDX_CHUNK)],
                        idx_v.at[c])
    copies = [
        pltpu.async_copy(table_hbm.at[idx_v.at[c]],
                         rows_v.at[pl.ds(c * _IDX_CHUNK, _IDX_CHUNK)], sem)
        for c in range(nchunks)
    ]
    for cp in copies:
        cp.wait()
    pltpu.sync_copy(rows_v, out_hbm.at[pl.ds(base, _ROWS_PER_W)])


def kernel(x, codebook):
    B, D, H, W = x.shape
    z2d = jnp.transpose(x, (0, 2, 3, 1)).reshape(_N, _D)
    cbt2 = -2.0 * codebook.T
    idx = _tc_argmin(z2d, cbt2)
    table128 = jnp.pad(codebook, ((0, 0), (0, _GATHER_W - _D)))
    zq = _sc_gather(table128, idx)[:, :_D]
    zq4 = jnp.transpose(zq.reshape(B, H, W, D), (0, 3, 1, 2))
    x_tilde = x + (zq4 - x)
    return (x_tilde, x, zq4)
